# baseline (speedup 1.0000x reference)
import functools
import math

import jax
import jax.numpy as jnp
from jax import lax
from jax.experimental import pallas as pl
from jax.experimental.pallas import tpu as pltpu
from jax.experimental.pallas import tpu_sc as plsc

_DIM = 64
_SCALE = math.sqrt(float(_DIM))

_NC = 2
_NS = 16
_NW = _NC * _NS

_V = 1000000
_VT = (_V + 127) // 128

_COMPACT = pltpu.CompilerParams(
    use_tc_tiling_on_sc=True, needs_layout_passes=False)


def _mesh():
  return plsc.VectorSubcoreMesh(core_axis_name="c", subcore_axis_name="s")


@functools.lru_cache(maxsize=None)
def _build_k1():
  nbuf = 2

  @functools.partial(
      pl.kernel,
      mesh=_mesh(),
      out_type=jax.ShapeDtypeStruct((_V // 2 + 32, 128), jnp.float32),
      scratch_types=(
          [pltpu.VMEM((_DIM, 128), jnp.float32)] * nbuf
          + [pltpu.VMEM((_DIM, 128), jnp.float32)] * nbuf
          + [pltpu.SemaphoreType.DMA] * (2 * nbuf)
      ),
      compiler_params=_COMPACT,
  )
  def k1(tt_hbm, out_hbm, v0, v1, o0, o1, gs0, gs1, os0, os1):
    vbuf = (v0, v1)
    obuf = (o0, o1)
    gsem = (gs0, gs1)
    osem = (os0, os1)
    wid = lax.axis_index("s") * _NC + lax.axis_index("c")
    n_c = (_VT - wid + _NW - 1) // _NW

    def fire_in(ci, p):
      c = wid + ci * _NW
      pltpu.async_copy(
          tt_hbm.at[pl.ds(0, _DIM), pl.ds(c * 128, 128)], vbuf[p], gsem[p])

    def wait_in(p):
      pltpu.make_async_copy(
          tt_hbm.at[pl.ds(0, _DIM), pl.ds(0, 128)], vbuf[p], gsem[p]).wait()

    def fire_out(ci, p):
      c = wid + ci * _NW
      pltpu.async_copy(
          obuf[p], out_hbm.at[pl.ds(c * 64, _DIM)], osem[p])

    def wait_out(p):
      pltpu.make_async_copy(
          obuf[p], out_hbm.at[pl.ds(0, _DIM)], osem[p]).wait()

    fire_in(0, 0)

    @pl.loop(0, n_c)
    def col_loop(ci):
      p = lax.rem(ci, 2)

      def with_bufs(p):
        vb, ob = vbuf[p], obuf[p]
        wait_in(p)

        @pl.when(ci + 1 < n_c)
        def _():
          fire_in(ci + 1, 1 - p)

        c = wid + ci * _NW
        vmax = jnp.where(c == _VT - 1, _V - (_VT - 1) * 128, 128)
        iot = lax.iota(jnp.int32, 16)

        @pl.loop(0, vmax)
        def v_loop(v):
          vb16 = jnp.full((16,), v, jnp.int32)
          half = lax.shift_right_logical(v, 1)
          col = (v & 1) * _DIM
          for cc in range(_DIM // 16):
            g16 = plsc.load_gather(vb, [cc * 16 + iot, vb16])
            ob[half, pl.ds(col + cc * 16, 16)] = g16 * _SCALE

        @pl.when(ci >= 2)
        def _():
          wait_out(p)

        fire_out(ci, p)

      lax.cond(p == 0, lambda: with_bufs(0), lambda: with_bufs(1))

    wait_out(0)
    wait_out(1)

  return k1


@functools.lru_cache(maxsize=None)
def _build_k2(n_t, n_b):
  assert n_b == _NW * 128
  t_tiles = n_t // 8
  assert t_tiles * 8 == n_t
  nbuf = 2

  @functools.partial(
      pl.kernel,
      mesh=_mesh(),
      out_type=jax.ShapeDtypeStruct((n_t, _DIM, n_b), jnp.float32),
      scratch_types=(
          [pltpu.VMEM((8, 128), jnp.int32)]
          + [pltpu.VMEM((8, 128), jnp.int32)]
          + [pltpu.VMEM((128, 128), jnp.float32)] * nbuf
          + [pltpu.VMEM((_DIM, 128), jnp.float32)] * nbuf
          + [pltpu.SemaphoreType.DMA] * (2 * nbuf)
      ),
      compiler_params=_COMPACT,
  )
  def k2(it_hbm, t2_hbm, out_hbm, ibuf, hbuf, g0, g1, o0, o1,
         gs0, gs1, os0, os1):
    gbuf = (g0, g1)
    obuf = (o0, o1)
    gsem = (gs0, gs1)
    osem = (os0, os1)
    wid = lax.axis_index("s") * _NC + lax.axis_index("c")
    b0 = wid * 128
    iot = lax.iota(jnp.int32, 16)

    def load_idx_tile(tt):
      pltpu.sync_copy(
          it_hbm.at[pl.ds(tt * 8, 8), pl.ds(b0, 128)], ibuf)
      for tr in range(8):
        for cc in range(8):
          sl = pl.ds(cc * 16, 16)
          hbuf[tr, sl] = lax.shift_right_logical(ibuf[tr, sl], 1)

    def fire_gather(tr, p):
      pltpu.async_copy(t2_hbm.at[hbuf.at[tr]], gbuf[p], gsem[p])

    def wait_gather(p):
      pltpu.make_async_copy(
          t2_hbm.at[pl.ds(0, 128)], gbuf[p], gsem[p]).wait()

    def fire_out(t, p):
      pltpu.async_copy(
          obuf[p], out_hbm.at[t, pl.ds(0, _DIM), pl.ds(b0, 128)], osem[p])

    def wait_out(p):
      pltpu.make_async_copy(
          obuf[p], out_hbm.at[0, pl.ds(0, _DIM), pl.ds(0, 128)],
          osem[p]).wait()

    def compute(tr, p):
      gb, ob = gbuf[p], obuf[p]
      for cc in range(8):
        hv = (ibuf[tr, pl.ds(cc * 16, 16)] & 1) * _DIM
        rows = cc * 16 + iot

        @pl.loop(0, _DIM, init_carry=(hv, rows), unroll=4)
        def j_loop(j, carry):
          hvc, rowsc = carry
          g16 = plsc.load_gather(gb, [rowsc, hvc + j])
          ob[j, pl.ds(cc * 16, 16)] = g16
          return carry

    @pl.loop(0, t_tiles)
    def tile_loop(tt):
      load_idx_tile(tt)
      fire_gather(0, 0)
      for tr in range(8):
        p = tr % 2
        wait_gather(p)
        if tr < 7:
          fire_gather(tr + 1, 1 - p)
        compute(tr, p)

        @pl.when((tt * 8 + tr) >= 2)
        def _():
          wait_out(p)

        fire_out(tt * 8 + tr, p)

    wait_out(0)
    wait_out(1)

  return k2


def kernel(inputs, table):
  b, t = inputs.shape
  tt = jnp.transpose(table)
  t2 = _build_k1()(tt)
  it = jnp.transpose(inputs).astype(jnp.int32)
  o3 = _build_k2(t, b)(it, t2)
  return jnp.transpose(o3, (2, 0, 1))


# speedup vs baseline: 1.9233x; 1.9233x over previous
import functools
import math

import jax
import jax.numpy as jnp
from jax import lax
from jax.experimental import pallas as pl
from jax.experimental.pallas import tpu as pltpu
from jax.experimental.pallas import tpu_sc as plsc

_DIM = 64
_SCALE = math.sqrt(float(_DIM))

_NC = 2
_NS = 16
_NW = _NC * _NS

_V = 1000000
_VT = (_V + 127) // 128

_COMPACT = pltpu.CompilerParams(
    use_tc_tiling_on_sc=True, needs_layout_passes=False)


def _mesh():
  return plsc.VectorSubcoreMesh(core_axis_name="c", subcore_axis_name="s")


@functools.lru_cache(maxsize=None)
def _build_k1():
  nbuf = 2

  @functools.partial(
      pl.kernel,
      mesh=_mesh(),
      out_type=jax.ShapeDtypeStruct((_V // 2 + 32, 128), jnp.float32),
      scratch_types=(
          [pltpu.VMEM((_DIM, 128), jnp.float32)] * nbuf
          + [pltpu.VMEM((_DIM, 128), jnp.float32)] * nbuf
          + [pltpu.SemaphoreType.DMA] * (2 * nbuf)
      ),
      compiler_params=_COMPACT,
  )
  def k1(tt_hbm, out_hbm, v0, v1, o0, o1, gs0, gs1, os0, os1):
    vbuf = (v0, v1)
    obuf = (o0, o1)
    gsem = (gs0, gs1)
    osem = (os0, os1)
    wid = lax.axis_index("s") * _NC + lax.axis_index("c")
    n_c = (_VT - wid + _NW - 1) // _NW

    def fire_in(ci, p):
      c = wid + ci * _NW
      pltpu.async_copy(
          tt_hbm.at[pl.ds(0, _DIM), pl.ds(c * 128, 128)], vbuf[p], gsem[p])

    def wait_in(p):
      pltpu.make_async_copy(
          tt_hbm.at[pl.ds(0, _DIM), pl.ds(0, 128)], vbuf[p], gsem[p]).wait()

    def fire_out(ci, p):
      c = wid + ci * _NW
      pltpu.async_copy(
          obuf[p], out_hbm.at[pl.ds(c * 64, _DIM)], osem[p])

    def wait_out(p):
      pltpu.make_async_copy(
          obuf[p], out_hbm.at[pl.ds(0, _DIM)], osem[p]).wait()

    fire_in(0, 0)

    @pl.loop(0, n_c)
    def col_loop(ci):
      p = lax.rem(ci, 2)

      def with_bufs(p):
        vb, ob = vbuf[p], obuf[p]
        wait_in(p)

        @pl.when(ci + 1 < n_c)
        def _():
          fire_in(ci + 1, 1 - p)

        iot = lax.iota(jnp.int32, 16)

        @plsc.parallel_loop(0, 128, unroll=8)
        def v_loop(v):
          vb16 = jnp.full((16,), v, jnp.int32)
          half = lax.shift_right_logical(v, 1)
          col = (v & 1) * _DIM
          for cc in range(_DIM // 16):
            g16 = plsc.load_gather(vb, [cc * 16 + iot, vb16])
            ob[half, pl.ds(col + cc * 16, 16)] = g16 * _SCALE

        @pl.when(ci >= 2)
        def _():
          wait_out(p)

        fire_out(ci, p)

      lax.cond(p == 0, lambda: with_bufs(0), lambda: with_bufs(1))

    wait_out(0)
    wait_out(1)

  return k1


@functools.lru_cache(maxsize=None)
def _build_k2(n_t, n_b):
  assert n_b == _NW * 128
  t_tiles = n_t // 8
  assert t_tiles * 8 == n_t
  nbuf = 2

  @functools.partial(
      pl.kernel,
      mesh=_mesh(),
      out_type=jax.ShapeDtypeStruct((n_t, _DIM, n_b), jnp.float32),
      scratch_types=(
          [pltpu.VMEM((8, 128), jnp.int32)]
          + [pltpu.VMEM((8, 128), jnp.int32)]
          + [pltpu.VMEM((128, 128), jnp.float32)] * nbuf
          + [pltpu.VMEM((_DIM, 128), jnp.float32)] * nbuf
          + [pltpu.SemaphoreType.DMA] * (2 * nbuf)
      ),
      compiler_params=_COMPACT,
  )
  def k2(it_hbm, t2_hbm, out_hbm, ibuf, hbuf, g0, g1, o0, o1,
         gs0, gs1, os0, os1):
    gbuf = (g0, g1)
    obuf = (o0, o1)
    gsem = (gs0, gs1)
    osem = (os0, os1)
    wid = lax.axis_index("s") * _NC + lax.axis_index("c")
    b0 = wid * 128
    iot = lax.iota(jnp.int32, 16)

    def load_idx_tile(tt):
      pltpu.sync_copy(
          it_hbm.at[pl.ds(tt * 8, 8), pl.ds(b0, 128)], ibuf)
      for tr in range(8):
        for cc in range(8):
          sl = pl.ds(cc * 16, 16)
          hbuf[tr, sl] = lax.shift_right_logical(ibuf[tr, sl], 1)

    def fire_gather(tr, p):
      pltpu.async_copy(t2_hbm.at[hbuf.at[tr]], gbuf[p], gsem[p])

    def wait_gather(p):
      pltpu.make_async_copy(
          t2_hbm.at[pl.ds(0, 128)], gbuf[p], gsem[p]).wait()

    def fire_out(t, p):
      pltpu.async_copy(
          obuf[p], out_hbm.at[t, pl.ds(0, _DIM), pl.ds(b0, 128)], osem[p])

    def wait_out(p):
      pltpu.make_async_copy(
          obuf[p], out_hbm.at[0, pl.ds(0, _DIM), pl.ds(0, 128)],
          osem[p]).wait()

    def compute(tr, p):
      gb, ob = gbuf[p], obuf[p]
      for cc in range(8):
        hv = (ibuf[tr, pl.ds(cc * 16, 16)] & 1) * _DIM
        rows = cc * 16 + iot

        @plsc.parallel_loop(0, _DIM, unroll=8, carry=(hv, rows))
        def j_loop(j, carry):
          hvc, rowsc = carry
          g16 = plsc.load_gather(gb, [rowsc, hvc + j])
          ob[j, pl.ds(cc * 16, 16)] = g16
          return carry

    @pl.loop(0, t_tiles)
    def tile_loop(tt):
      load_idx_tile(tt)
      fire_gather(0, 0)
      for tr in range(8):
        p = tr % 2
        wait_gather(p)
        if tr < 7:
          fire_gather(tr + 1, 1 - p)
        compute(tr, p)

        @pl.when((tt * 8 + tr) >= 2)
        def _():
          wait_out(p)

        fire_out(tt * 8 + tr, p)

    wait_out(0)
    wait_out(1)

  return k2


def kernel(inputs, table):
  b, t = inputs.shape
  tt = jnp.transpose(table)
  t2 = _build_k1()(tt)
  it = jnp.transpose(inputs).astype(jnp.int32)
  o3 = _build_k2(t, b)(it, t2)
  return jnp.transpose(o3, (2, 0, 1))
